# Initial kernel scaffold; baseline (speedup 1.0000x reference)
import jax
import jax.numpy as jnp
from jax.experimental import pallas as pl


def kernel(m_logits):
    raise NotImplementedError("write your pallas kernel here")


# speedup vs baseline: 1.3301x; 1.3301x over previous
import functools

import jax
import jax.numpy as jnp
from jax import lax
from jax.experimental import pallas as pl
from jax.experimental.pallas import tpu as pltpu
from jax.experimental.pallas import tpu_sc as plsc

R = 64
V = 100000
NC = 2
NS = 16
NW = NC * NS
RPW = R // NW
L = 16

CHUNK = 20000
NCHUNK = V // CHUNK
U = 5
VPC = CHUNK // L
OUTER = VPC // U

NEG_INF = float("-inf")
I32_MAX = 2**31 - 1


def _bcast_max(x):
    c = plsc.cummax(x)
    return plsc.cummax(lax.rev(c, (0,)))


def _greedy_body(m_hbm, out_hbm, buf0, buf1, res_v, sem0, sem1):
    wid = lax.axis_index("c") * NS + lax.axis_index("s")
    bufs = (buf0, buf1)
    sems = (sem0, sem1)
    lane = lax.iota(jnp.int32, L)

    tasks = [(r, c) for r in range(RPW) for c in range(NCHUNK)]

    def start(t):
        r, c = tasks[t]
        base = (wid * RPW + r) * V + c * CHUNK
        cp = pltpu.make_async_copy(
            m_hbm.at[pl.ds(base, CHUNK)], bufs[t % 2], sems[t % 2]
        )
        cp.start()
        return cp

    inflight = {0: start(0)}

    results = []
    for r in range(RPW):
        accs = [(jnp.full((L,), NEG_INF, jnp.float32),
                 jnp.zeros((L,), jnp.int32))
                for _ in range(U)]
        for c in range(NCHUNK):
            t = r * NCHUNK + c
            if t + 1 < len(tasks):
                inflight[t + 1] = start(t + 1)
            inflight.pop(t).wait()
            buf = bufs[t % 2]

            def body(i, carry, _buf=buf, _c=c):
                bi = jnp.full((L,), jnp.int32(0)) + (i + _c * OUTER)
                new = []
                for j, (av, ai) in enumerate(carry):
                    v = _buf[pl.ds((i * U + j) * L, L)]
                    p = v > av
                    new.append((jnp.where(p, v, av), jnp.where(p, bi, ai)))
                return tuple(new)

            accs = lax.fori_loop(0, OUTER, body, tuple(accs))

        cand = []
        for j, (av, ai) in enumerate(accs):
            gidx = (ai * U + j) * L + lane
            cand.append((av, gidx))
        while len(cand) > 1:
            nxt = []
            for k in range(0, len(cand) - 1, 2):
                (va, ia), (vb, ib) = cand[k], cand[k + 1]
                p = (va > vb) | ((va == vb) & (ia < ib))
                nxt.append((jnp.where(p, va, vb), jnp.where(p, ia, ib)))
            if len(cand) % 2:
                nxt.append(cand[-1])
            cand = nxt
        av, ai = cand[0]
        rmax = _bcast_max(av)
        masked = jnp.where(av == rmax, ai, jnp.int32(I32_MAX))
        results.append(-_bcast_max(-masked))

    out_vec = jnp.full((L,), jnp.int32(0))
    for r in range(RPW):
        out_vec = jnp.where(lane == r, results[r], out_vec)
    res_v[...] = out_vec
    pltpu.sync_copy(res_v, out_hbm.at[wid])


@functools.partial(jax.jit, donate_argnums=())
def kernel(m_logits):
    m_flat = m_logits.reshape(R * V)
    staged = pl.kernel(
        _greedy_body,
        out_type=jax.ShapeDtypeStruct((NW, L), jnp.int32),
        mesh=plsc.VectorSubcoreMesh(core_axis_name="c", subcore_axis_name="s"),
        scratch_types=[
            pltpu.VMEM((CHUNK,), jnp.float32),
            pltpu.VMEM((CHUNK,), jnp.float32),
            pltpu.VMEM((L,), jnp.int32),
            pltpu.SemaphoreType.DMA,
            pltpu.SemaphoreType.DMA,
        ],
        compiler_params=pltpu.CompilerParams(needs_layout_passes=False),
        name="greedy_head_sc",
    )(m_flat)
    return staged[:, :RPW].reshape(R, 1)


# speedup vs baseline: 1.7808x; 1.3388x over previous
import functools

import jax
import jax.numpy as jnp
from jax import lax
from jax.experimental import pallas as pl
from jax.experimental.pallas import tpu as pltpu
from jax.experimental.pallas import tpu_sc as plsc

R = 64
V = 100000
NC = 2
NS = 16
NW = NC * NS
L = 16
U = 5

GROUPS = 8
QUARTERS = 4
QW = 24960
CW = 2560
EDGE_T = 781 * 128
EDGEW = V - EDGE_T

NEG_INF = float("-inf")
I32_MAX = 2**31 - 1


def _bcast_max(x):
    c = plsc.cummax(x)
    return plsc.cummax(lax.rev(c, (0,)))


def _merge(a, b):
    (va, ia), (vb, ib) = a, b
    p = (va > vb) | ((va == vb) & (ia < ib))
    return jnp.where(p, va, vb), jnp.where(p, ia, ib)


def _greedy_body(m_hbm, edge_hbm, out_hbm, stv_hbm, sti_hbm, buf0, buf1,
                 bufe, resv_v, resi_v, lv_v, li_v, sem0, sem1, seme):
    wid = lax.axis_index("c") * NS + lax.axis_index("s")
    g = wid // QUARTERS
    q = wid % QUARTERS
    row0 = pl.multiple_of(g * 8, 8)
    qbase = pl.multiple_of(q * QW, 128)
    lane = lax.iota(jnp.int32, L)
    sems = (sem0, sem1)

    qend = jnp.where(q == QUARTERS - 1, EDGE_T, (q + 1) * QW)
    chunks = [(buf0 if k % 2 == 0 else buf1,
               qbase + k * CW if k < 9 else qend - CW)
              for k in range(10)]

    def start(t):
        buf, col = chunks[t]
        cp = pltpu.make_async_copy(
            m_hbm.at[pl.ds(row0, 8), pl.ds(pl.multiple_of(col, 128), CW)],
            buf,
            sems[t % 2],
        )
        cp.start()
        return cp

    inflight = {0: start(0)}
    edge_cp = pltpu.make_async_copy(edge_hbm.at[pl.ds(row0, 8), :], bufe,
                                    seme)
    edge_cp.start()

    persist = [(jnp.full((L,), NEG_INF, jnp.float32),
                jnp.zeros((L,), jnp.int32)) for _ in range(8)]

    outer = CW // (U * L)
    for t, (buf, col) in enumerate(chunks):
        if t + 1 < len(chunks):
            inflight[t + 1] = start(t + 1)
        inflight.pop(t).wait()

        for r in range(8):
            def body(i, carry, _buf=buf, _r=r):
                bi = jnp.zeros((L,), jnp.int32) + i
                new = []
                for j, (av, ai) in enumerate(carry):
                    v = _buf[_r, pl.ds((i * U + j) * L, L)]
                    p = v > av
                    new.append((jnp.where(p, v, av), jnp.where(p, bi, ai)))
                return tuple(new)

            accs = lax.fori_loop(
                0, outer,
                body,
                tuple((jnp.full((L,), NEG_INF, jnp.float32),
                       jnp.zeros((L,), jnp.int32)) for _ in range(U)),
            )
            cand = [(av, ai * (U * L) + (col + j * L) + lane)
                    for j, (av, ai) in enumerate(accs)]
            while len(cand) > 1:
                nxt = [_merge(cand[k], cand[k + 1])
                       for k in range(0, len(cand) - 1, 2)]
                if len(cand) % 2:
                    nxt.append(cand[-1])
                cand = nxt
            pv, pi = persist[r]
            cv, ci = cand[0]
            p = cv > pv
            persist[r] = (jnp.where(p, cv, pv), jnp.where(p, ci, pi))

    edge_cp.wait()
    for r in range(8):
        pv, pi = persist[r]
        for k in range(EDGEW // L):
            v = bufe[r, pl.ds(k * L, L)]
            gi = (EDGE_T + k * L) + lane
            p = v > pv
            pv, pi = jnp.where(p, v, pv), jnp.where(p, gi, pi)
        persist[r] = (pv, pi)

    res_val = jnp.zeros((L,), jnp.float32)
    res_idx = jnp.zeros((L,), jnp.int32)
    for r in range(8):
        pv, pi = persist[r]
        rmax = _bcast_max(pv)
        masked = jnp.where(pv == rmax, pi, jnp.int32(I32_MAX))
        ridx = -_bcast_max(-masked)
        res_val = jnp.where(lane == r, rmax, res_val)
        res_idx = jnp.where(lane == r, ridx, res_idx)

    resv_v[...] = res_val
    resi_v[...] = res_idx
    pltpu.sync_copy(resv_v, stv_hbm.at[pl.ds(wid * L, L)])
    pltpu.sync_copy(resi_v, sti_hbm.at[pl.ds(wid * L, L)])
    plsc.subcore_barrier()

    @pl.when(q == 0)
    def _():
        pltpu.sync_copy(stv_hbm.at[pl.ds(wid * L, QUARTERS * L)], lv_v)
        pltpu.sync_copy(sti_hbm.at[pl.ds(wid * L, QUARTERS * L)], li_v)
        best = (lv_v[pl.ds(0, L)], li_v[pl.ds(0, L)])
        for k in range(1, QUARTERS):
            best = _merge(best, (lv_v[pl.ds(k * L, L)],
                                 li_v[pl.ds(k * L, L)]))
        resi_v[...] = best[1]
        pltpu.sync_copy(resi_v, out_hbm.at[g])


@functools.partial(jax.jit, donate_argnums=())
def kernel(m_logits):
    staged, _, _ = pl.kernel(
        _greedy_body,
        out_type=(jax.ShapeDtypeStruct((GROUPS, L), jnp.int32),
                  jax.ShapeDtypeStruct((NW * L,), jnp.float32),
                  jax.ShapeDtypeStruct((NW * L,), jnp.int32)),
        mesh=plsc.VectorSubcoreMesh(core_axis_name="c", subcore_axis_name="s"),
        scratch_types=[
            pltpu.VMEM((8, CW), jnp.float32),
            pltpu.VMEM((8, CW), jnp.float32),
            pltpu.VMEM((8, EDGEW), jnp.float32),
            pltpu.VMEM((L,), jnp.float32),
            pltpu.VMEM((L,), jnp.int32),
            pltpu.VMEM((QUARTERS * L,), jnp.float32),
            pltpu.VMEM((QUARTERS * L,), jnp.int32),
            pltpu.SemaphoreType.DMA,
            pltpu.SemaphoreType.DMA,
            pltpu.SemaphoreType.DMA,
        ],
        compiler_params=pltpu.CompilerParams(needs_layout_passes=False),
        name="greedy_head_sc",
    )(m_logits, m_logits[:, EDGE_T:])
    return staged[:, :8].reshape(R, 1)


# speedup vs baseline: 2.2039x; 1.2376x over previous
import functools

import jax
import jax.numpy as jnp
from jax import lax
from jax.experimental import pallas as pl
from jax.experimental.pallas import tpu as pltpu
from jax.experimental.pallas import tpu_sc as plsc

R = 64
V = 100000
NC = 2
NS = 16
NW = NC * NS
L = 16
U = 5

GROUPS = 8
QUARTERS = 4
QW = 24960
CW = 2560
NCHUNK = 10
EDGE_T = 781 * 128
EDGEW = V - EDGE_T

NEG_INF = float("-inf")
I32_MAX = 2**31 - 1


def _bcast_max(x):
    c = plsc.cummax(x)
    return plsc.cummax(lax.rev(c, (0,)))


def _merge(a, b):
    (va, ia), (vb, ib) = a, b
    p = (va > vb) | ((va == vb) & (ia < ib))
    return jnp.where(p, va, vb), jnp.where(p, ia, ib)


def _greedy_body(m_hbm, edge_hbm, out_hbm, stv_hbm, sti_hbm, buf0, buf1,
                 bufe, pv_v, pi_v, resv_v, resi_v, lv_v, li_v,
                 sem0, sem1, seme):
    wid = lax.axis_index("c") * NS + lax.axis_index("s")
    g = wid // QUARTERS
    q = wid % QUARTERS
    row0 = pl.multiple_of(g * 8, 8)
    qbase = pl.multiple_of(q * QW, 128)
    lane = lax.iota(jnp.int32, L)
    sems = (sem0, sem1)

    qend = jnp.where(q == QUARTERS - 1, EDGE_T, (q + 1) * QW)

    def chunk_col(t):
        return pl.multiple_of(
            jnp.where(t == NCHUNK - 1, qend - CW, qbase + t * CW), 128)

    def start(t, buf, sem):
        pltpu.make_async_copy(
            m_hbm.at[pl.ds(row0, 8), pl.ds(chunk_col(t), CW)], buf, sem
        ).start()

    start(0, buf0, sem0)
    start(1, buf1, sem1)
    edge_cp = pltpu.make_async_copy(edge_hbm.at[pl.ds(row0, 8), :], bufe,
                                    seme)
    edge_cp.start()

    def init_row(r, _):
        pv_v[r, :] = jnp.full((L,), NEG_INF, jnp.float32)
        pi_v[r, :] = jnp.zeros((L,), jnp.int32)
        return 0

    lax.fori_loop(0, 8, init_row, 0)

    outer = CW // (U * L)

    def process(t, buf):
        col = chunk_col(t)

        def row_body(r, _):
            def body(i, carry):
                bi = jnp.zeros((L,), jnp.int32) + i
                new = []
                for j, (av, ai) in enumerate(carry):
                    v = buf[r, pl.ds((i * U + j) * L, L)]
                    p = v > av
                    new.append((jnp.where(p, v, av), jnp.where(p, bi, ai)))
                return tuple(new)

            accs = lax.fori_loop(
                0, outer,
                body,
                tuple((jnp.full((L,), NEG_INF, jnp.float32),
                       jnp.zeros((L,), jnp.int32)) for _ in range(U)),
            )
            cand = [(av, ai * (U * L) + (col + j * L) + lane)
                    for j, (av, ai) in enumerate(accs)]
            while len(cand) > 1:
                nxt = [_merge(cand[k], cand[k + 1])
                       for k in range(0, len(cand) - 1, 2)]
                if len(cand) % 2:
                    nxt.append(cand[-1])
                cand = nxt
            pv, pi = pv_v[r, :], pi_v[r, :]
            cv, ci = cand[0]
            p = cv > pv
            pv_v[r, :] = jnp.where(p, cv, pv)
            pi_v[r, :] = jnp.where(p, ci, pi)
            return 0

        lax.fori_loop(0, 8, row_body, 0)

    def wait(t, buf, sem):
        pltpu.make_async_copy(
            m_hbm.at[pl.ds(row0, 8), pl.ds(chunk_col(t), CW)], buf, sem
        ).wait()

    def pair_body(p, _):
        t0 = p * 2
        wait(t0, buf0, sem0)
        process(t0, buf0)

        @pl.when(t0 + 2 < NCHUNK)
        def _():
            start(t0 + 2, buf0, sem0)

        wait(t0 + 1, buf1, sem1)
        process(t0 + 1, buf1)

        @pl.when(t0 + 3 < NCHUNK)
        def _():
            start(t0 + 3, buf1, sem1)

        return 0

    lax.fori_loop(0, NCHUNK // 2, pair_body, 0)

    edge_cp.wait()

    res_val = jnp.zeros((L,), jnp.float32)
    res_idx = jnp.zeros((L,), jnp.int32)
    for r in range(8):
        pv, pi = pv_v[r, :], pi_v[r, :]
        for k in range(EDGEW // L):
            v = bufe[r, pl.ds(k * L, L)]
            gi = (EDGE_T + k * L) + lane
            p = v > pv
            pv, pi = jnp.where(p, v, pv), jnp.where(p, gi, pi)
        rmax = _bcast_max(pv)
        masked = jnp.where(pv == rmax, pi, jnp.int32(I32_MAX))
        ridx = -_bcast_max(-masked)
        res_val = jnp.where(lane == r, rmax, res_val)
        res_idx = jnp.where(lane == r, ridx, res_idx)

    resv_v[...] = res_val
    resi_v[...] = res_idx
    pltpu.sync_copy(resv_v, stv_hbm.at[pl.ds(wid * L, L)])
    pltpu.sync_copy(resi_v, sti_hbm.at[pl.ds(wid * L, L)])
    plsc.subcore_barrier()

    @pl.when(q == 0)
    def _():
        pltpu.sync_copy(stv_hbm.at[pl.ds(wid * L, QUARTERS * L)], lv_v)
        pltpu.sync_copy(sti_hbm.at[pl.ds(wid * L, QUARTERS * L)], li_v)
        best = (lv_v[pl.ds(0, L)], li_v[pl.ds(0, L)])
        for k in range(1, QUARTERS):
            best = _merge(best, (lv_v[pl.ds(k * L, L)],
                                 li_v[pl.ds(k * L, L)]))
        resi_v[...] = best[1]
        pltpu.sync_copy(resi_v, out_hbm.at[g])


@functools.partial(jax.jit, donate_argnums=())
def kernel(m_logits):
    staged, _, _ = pl.kernel(
        _greedy_body,
        out_type=(jax.ShapeDtypeStruct((GROUPS, L), jnp.int32),
                  jax.ShapeDtypeStruct((NW * L,), jnp.float32),
                  jax.ShapeDtypeStruct((NW * L,), jnp.int32)),
        mesh=plsc.VectorSubcoreMesh(core_axis_name="c", subcore_axis_name="s"),
        scratch_types=[
            pltpu.VMEM((8, CW), jnp.float32),
            pltpu.VMEM((8, CW), jnp.float32),
            pltpu.VMEM((8, EDGEW), jnp.float32),
            pltpu.VMEM((8, L), jnp.float32),
            pltpu.VMEM((8, L), jnp.int32),
            pltpu.VMEM((L,), jnp.float32),
            pltpu.VMEM((L,), jnp.int32),
            pltpu.VMEM((QUARTERS * L,), jnp.float32),
            pltpu.VMEM((QUARTERS * L,), jnp.int32),
            pltpu.SemaphoreType.DMA,
            pltpu.SemaphoreType.DMA,
            pltpu.SemaphoreType.DMA,
        ],
        compiler_params=pltpu.CompilerParams(needs_layout_passes=False),
        name="greedy_head_sc",
    )(m_logits, m_logits[:, EDGE_T:])
    return staged[:, :8].reshape(R, 1)


# speedup vs baseline: 2.5018x; 1.1351x over previous
import functools

import jax
import jax.numpy as jnp
from jax import lax
from jax.experimental import pallas as pl
from jax.experimental.pallas import tpu as pltpu
from jax.experimental.pallas import tpu_sc as plsc

R = 64
V = 100000
NC = 2
NS = 16
NW = NC * NS
L = 16
U = 5

GROUPS = 8
QUARTERS = 4
QW = 24960
CW = 2560
NCHUNK = 10
EDGE_T = 781 * 128
EDGEW = V - EDGE_T

NEG_INF = float("-inf")
I32_MAX = 2**31 - 1


def _bcast_max(x):
    c = plsc.cummax(x)
    return plsc.cummax(lax.rev(c, (0,)))


def _merge(a, b):
    (va, ia), (vb, ib) = a, b
    p = (va > vb) | ((va == vb) & (ia < ib))
    return jnp.where(p, va, vb), jnp.where(p, ia, ib)


def _greedy_body(m_hbm, edge_hbm, out_hbm, stv_hbm, sti_hbm, buf0, buf1,
                 bufe, resv_v, resi_v, lv_v, li_v, sem0, sem1, seme):
    wid = lax.axis_index("c") * NS + lax.axis_index("s")
    g = wid // QUARTERS
    q = wid % QUARTERS
    row0 = pl.multiple_of(g * 8, 8)
    qbase = pl.multiple_of(q * QW, 128)
    lane = lax.iota(jnp.int32, L)
    sems = (sem0, sem1)

    qend = jnp.where(q == QUARTERS - 1, EDGE_T, (q + 1) * QW)

    def chunk_col(t):
        return pl.multiple_of(
            jnp.where(t == NCHUNK - 1, qend - CW, qbase + t * CW), 128)

    def start(t, buf, sem):
        pltpu.make_async_copy(
            m_hbm.at[pl.ds(row0, 8), pl.ds(chunk_col(t), CW)], buf, sem
        ).start()

    start(0, buf0, sem0)
    start(1, buf1, sem1)
    edge_cp = pltpu.make_async_copy(edge_hbm.at[pl.ds(row0, 8), :], bufe,
                                    seme)
    edge_cp.start()

    outer = CW // L

    def process(t, buf, accs):
        cbase = chunk_col(t) // L

        def body(i, carry):
            bi = jnp.zeros((L,), jnp.int32) + (cbase + i)
            new = []
            for r, (av, ai) in enumerate(carry):
                v = buf[r, pl.ds(i * L, L)]
                p = v > av
                new.append((jnp.where(p, v, av), jnp.where(p, bi, ai)))
            return tuple(new)

        return lax.fori_loop(0, outer, body, accs)

    def wait(t, buf, sem):
        pltpu.make_async_copy(
            m_hbm.at[pl.ds(row0, 8), pl.ds(chunk_col(t), CW)], buf, sem
        ).wait()

    def pair_body(p, accs):
        t0 = p * 2
        wait(t0, buf0, sem0)
        accs = process(t0, buf0, accs)

        @pl.when(t0 + 2 < NCHUNK)
        def _():
            start(t0 + 2, buf0, sem0)

        wait(t0 + 1, buf1, sem1)
        accs = process(t0 + 1, buf1, accs)

        @pl.when(t0 + 3 < NCHUNK)
        def _():
            start(t0 + 3, buf1, sem1)

        return accs

    accs = tuple((jnp.full((L,), NEG_INF, jnp.float32),
                  jnp.zeros((L,), jnp.int32)) for _ in range(8))
    accs = lax.fori_loop(0, NCHUNK // 2, pair_body, accs)

    edge_cp.wait()
    res_val = jnp.zeros((L,), jnp.float32)
    res_idx = jnp.zeros((L,), jnp.int32)
    for r in range(8):
        pv, ai = accs[r]
        for k in range(EDGEW // L):
            v = bufe[r, pl.ds(k * L, L)]
            bi = jnp.zeros((L,), jnp.int32) + (EDGE_T // L + k)
            p = v > pv
            pv, ai = jnp.where(p, v, pv), jnp.where(p, bi, ai)
        pi = ai * L + lane
        rmax = _bcast_max(pv)
        masked = jnp.where(pv == rmax, pi, jnp.int32(I32_MAX))
        ridx = -_bcast_max(-masked)
        res_val = jnp.where(lane == r, rmax, res_val)
        res_idx = jnp.where(lane == r, ridx, res_idx)

    resv_v[...] = res_val
    resi_v[...] = res_idx
    pltpu.sync_copy(resv_v, stv_hbm.at[pl.ds(wid * L, L)])
    pltpu.sync_copy(resi_v, sti_hbm.at[pl.ds(wid * L, L)])
    plsc.subcore_barrier()

    @pl.when(q == 0)
    def _():
        pltpu.sync_copy(stv_hbm.at[pl.ds(wid * L, QUARTERS * L)], lv_v)
        pltpu.sync_copy(sti_hbm.at[pl.ds(wid * L, QUARTERS * L)], li_v)
        best = (lv_v[pl.ds(0, L)], li_v[pl.ds(0, L)])
        for k in range(1, QUARTERS):
            best = _merge(best, (lv_v[pl.ds(k * L, L)],
                                 li_v[pl.ds(k * L, L)]))
        resi_v[...] = best[1]
        pltpu.sync_copy(resi_v, out_hbm.at[g])


@functools.partial(jax.jit, donate_argnums=())
def kernel(m_logits):
    staged, _, _ = pl.kernel(
        _greedy_body,
        out_type=(jax.ShapeDtypeStruct((GROUPS, L), jnp.int32),
                  jax.ShapeDtypeStruct((NW * L,), jnp.float32),
                  jax.ShapeDtypeStruct((NW * L,), jnp.int32)),
        mesh=plsc.VectorSubcoreMesh(core_axis_name="c", subcore_axis_name="s"),
        scratch_types=[
            pltpu.VMEM((8, CW), jnp.float32),
            pltpu.VMEM((8, CW), jnp.float32),
            pltpu.VMEM((8, EDGEW), jnp.float32),
            pltpu.VMEM((L,), jnp.float32),
            pltpu.VMEM((L,), jnp.int32),
            pltpu.VMEM((QUARTERS * L,), jnp.float32),
            pltpu.VMEM((QUARTERS * L,), jnp.int32),
            pltpu.SemaphoreType.DMA,
            pltpu.SemaphoreType.DMA,
            pltpu.SemaphoreType.DMA,
        ],
        compiler_params=pltpu.CompilerParams(needs_layout_passes=False),
        name="greedy_head_sc",
    )(m_logits, m_logits[:, EDGE_T:])
    return staged[:, :8].reshape(R, 1)


# speedup vs baseline: 2.5328x; 1.0124x over previous
import functools

import jax
import jax.numpy as jnp
from jax import lax
from jax.experimental import pallas as pl
from jax.experimental.pallas import tpu as pltpu
from jax.experimental.pallas import tpu_sc as plsc

R = 64
V = 100000
NC = 2
NS = 16
NW = NC * NS
L = 16
U = 5

GROUPS = 8
QUARTERS = 4
QW = 24960
CW = 3200
NCHUNK = 8
EDGE_T = 781 * 128
EDGEW = V - EDGE_T

NEG_INF = float("-inf")
I32_MAX = 2**31 - 1


def _bcast_max(x):
    c = plsc.cummax(x)
    return plsc.cummax(lax.rev(c, (0,)))


def _merge(a, b):
    (va, ia), (vb, ib) = a, b
    p = (va > vb) | ((va == vb) & (ia < ib))
    return jnp.where(p, va, vb), jnp.where(p, ia, ib)


def _greedy_body(m_hbm, edge_hbm, out_hbm, stv_hbm, sti_hbm, buf0, buf1,
                 bufe, resv_v, resi_v, lv_v, li_v, sem0, sem1, seme):
    wid = lax.axis_index("c") * NS + lax.axis_index("s")
    g = wid // QUARTERS
    q = wid % QUARTERS
    row0 = pl.multiple_of(g * 8, 8)
    qbase = pl.multiple_of(q * QW, 128)
    lane = lax.iota(jnp.int32, L)
    sems = (sem0, sem1)

    qend = jnp.where(q == QUARTERS - 1, EDGE_T, (q + 1) * QW)

    def chunk_col(t):
        return pl.multiple_of(
            jnp.where(t == NCHUNK - 1, qend - CW, qbase + t * CW), 128)

    def start(t, buf, sem):
        pltpu.make_async_copy(
            m_hbm.at[pl.ds(row0, 8), pl.ds(chunk_col(t), CW)], buf, sem
        ).start()

    start(0, buf0, sem0)
    start(1, buf1, sem1)
    edge_cp = pltpu.make_async_copy(edge_hbm.at[pl.ds(row0, 8), :], bufe,
                                    seme)
    edge_cp.start()

    outer = CW // L

    def process(t, buf, accs):
        cbase = chunk_col(t) // L

        def body(i, carry):
            bi = jnp.zeros((L,), jnp.int32) + (cbase + i)
            new = []
            for r, (av, ai) in enumerate(carry):
                v = buf[r, pl.ds(i * L, L)]
                p = v > av
                new.append((jnp.where(p, v, av), jnp.where(p, bi, ai)))
            return tuple(new)

        return lax.fori_loop(0, outer, body, accs)

    def wait(t, buf, sem):
        pltpu.make_async_copy(
            m_hbm.at[pl.ds(row0, 8), pl.ds(chunk_col(t), CW)], buf, sem
        ).wait()

    def pair_body(p, accs):
        t0 = p * 2
        wait(t0, buf0, sem0)
        accs = process(t0, buf0, accs)

        @pl.when(t0 + 2 < NCHUNK)
        def _():
            start(t0 + 2, buf0, sem0)

        wait(t0 + 1, buf1, sem1)
        accs = process(t0 + 1, buf1, accs)

        @pl.when(t0 + 3 < NCHUNK)
        def _():
            start(t0 + 3, buf1, sem1)

        return accs

    accs = tuple((jnp.full((L,), NEG_INF, jnp.float32),
                  jnp.zeros((L,), jnp.int32)) for _ in range(8))
    accs = lax.fori_loop(0, NCHUNK // 2, pair_body, accs)

    edge_cp.wait()
    res_val = jnp.zeros((L,), jnp.float32)
    res_idx = jnp.zeros((L,), jnp.int32)
    for r in range(8):
        pv, ai = accs[r]
        for k in range(EDGEW // L):
            v = bufe[r, pl.ds(k * L, L)]
            bi = jnp.zeros((L,), jnp.int32) + (EDGE_T // L + k)
            p = v > pv
            pv, ai = jnp.where(p, v, pv), jnp.where(p, bi, ai)
        pi = ai * L + lane
        rmax = _bcast_max(pv)
        masked = jnp.where(pv == rmax, pi, jnp.int32(I32_MAX))
        ridx = -_bcast_max(-masked)
        res_val = jnp.where(lane == r, rmax, res_val)
        res_idx = jnp.where(lane == r, ridx, res_idx)

    resv_v[...] = res_val
    resi_v[...] = res_idx
    pltpu.sync_copy(resv_v, stv_hbm.at[pl.ds(wid * L, L)])
    pltpu.sync_copy(resi_v, sti_hbm.at[pl.ds(wid * L, L)])
    plsc.subcore_barrier()

    @pl.when(q == 0)
    def _():
        pltpu.sync_copy(stv_hbm.at[pl.ds(wid * L, QUARTERS * L)], lv_v)
        pltpu.sync_copy(sti_hbm.at[pl.ds(wid * L, QUARTERS * L)], li_v)
        best = (lv_v[pl.ds(0, L)], li_v[pl.ds(0, L)])
        for k in range(1, QUARTERS):
            best = _merge(best, (lv_v[pl.ds(k * L, L)],
                                 li_v[pl.ds(k * L, L)]))
        resi_v[...] = best[1]
        pltpu.sync_copy(resi_v, out_hbm.at[g])


@functools.partial(jax.jit, donate_argnums=())
def kernel(m_logits):
    staged, _, _ = pl.kernel(
        _greedy_body,
        out_type=(jax.ShapeDtypeStruct((GROUPS, L), jnp.int32),
                  jax.ShapeDtypeStruct((NW * L,), jnp.float32),
                  jax.ShapeDtypeStruct((NW * L,), jnp.int32)),
        mesh=plsc.VectorSubcoreMesh(core_axis_name="c", subcore_axis_name="s"),
        scratch_types=[
            pltpu.VMEM((8, CW), jnp.float32),
            pltpu.VMEM((8, CW), jnp.float32),
            pltpu.VMEM((8, EDGEW), jnp.float32),
            pltpu.VMEM((L,), jnp.float32),
            pltpu.VMEM((L,), jnp.int32),
            pltpu.VMEM((QUARTERS * L,), jnp.float32),
            pltpu.VMEM((QUARTERS * L,), jnp.int32),
            pltpu.SemaphoreType.DMA,
            pltpu.SemaphoreType.DMA,
            pltpu.SemaphoreType.DMA,
        ],
        compiler_params=pltpu.CompilerParams(needs_layout_passes=False),
        name="greedy_head_sc",
    )(m_logits, m_logits[:, EDGE_T:])
    return staged[:, :8].reshape(R, 1)
